# baseline (speedup 1.0000x reference)
import jax
import jax.numpy as jnp
from jax import lax
from jax.experimental import pallas as pl
from jax.experimental.pallas import tpu as pltpu
from jax.experimental.pallas import tpu_sc as plsc

N_NODES = 10000
N_PAD = 10240
N_EDGES = 320000
D_FEAT = 128
EB = N_EDGES // 128

NUM_CORES = 2
NUM_SUBCORES = 16
NW = NUM_CORES * NUM_SUBCORES
BPC = 4
CHUNK = BPC * 128
N_CHUNKS = EB // BPC


def _mm_body(x_ref, w_ref, o_ref):
    xb = x_ref[:]
    w = w_ref[:]
    xp = jnp.concatenate(
        [xb, jnp.zeros((N_PAD - N_NODES, D_FEAT), jnp.float32)], axis=0
    )
    dn = (((1,), (1,)), ((), ()))
    t1 = lax.dot_general(w[:, :D_FEAT], xp, dn, preferred_element_type=jnp.float32)
    t2 = lax.dot_general(w[:, D_FEAT:], xp, dn, preferred_element_type=jnp.float32)
    o_ref[:] = jnp.concatenate([t1, t2], axis=0)


_mm_call = pl.pallas_call(
    _mm_body,
    out_shape=jax.ShapeDtypeStruct((8, N_PAD), jnp.float32),
)


MAXCH = -(-N_CHUNKS // NW)


def _sc_body(
    table_hbm, q_hbm, p_hbm, table_v, idx0, idx1, out0, out1, si0, si1, so0, so1
):
    wid = lax.axis_index("s") * NUM_CORES + lax.axis_index("c")
    idx = (idx0, idx1)
    out = (out0, out1)
    si = (si0, si1)
    so = (so0, so1)

    pltpu.sync_copy(table_hbm, table_v)

    def q_slice(ci):
        return q_hbm.at[pl.ds(ci * BPC, BPC)]

    def p_slice(ci):
        return p_hbm.at[:, pl.ds(ci * BPC, BPC), :, :]

    def fire_idx(ci, b):
        @pl.when(ci < N_CHUNKS)
        def _():
            pltpu.async_copy(q_slice(ci), idx[b], si[b])

    def compute(b):
        @plsc.parallel_loop(0, CHUNK // 16, unroll=8)
        def _(g):
            eb = g // 8
            o = (g % 8) * 16
            r = idx[b][eb, 0, pl.ds(o, 16)]
            c = idx[b][eb, 1, pl.ds(o, 16)]
            for j in range(4):
                yj = plsc.load_gather(table_v, [jnp.full((16,), j, jnp.int32), r])
                zj = plsc.load_gather(table_v, [jnp.full((16,), j + 4, jnp.int32), c])
                s = yj + zj
                t = 1.0 - 2.0 / (jnp.exp(2.0 * s) + 1.0)
                out[b][j // 2, eb, j % 2, pl.ds(o, 16)] = t

    fire_idx(wid, 0)
    fire_idx(wid + NW, 1)

    def pair_body(i, carry):
        for b in range(2):
            ch = 2 * i + b
            ci = wid + ch * NW
            ci_prev = ci - 2 * NW

            @pl.when(jnp.logical_and(ch >= 2, ci_prev < N_CHUNKS))
            def _():
                pltpu.make_async_copy(out[b], p_slice(ci_prev), so[b]).wait()

            @pl.when(ci < N_CHUNKS)
            def _():
                pltpu.make_async_copy(q_slice(ci), idx[b], si[b]).wait()
                compute(b)
                pltpu.async_copy(out[b], p_slice(ci), so[b])

            fire_idx(ci + 2 * NW, b)
        return carry

    lax.fori_loop(0, MAXCH // 2, pair_body, 0)

    for ch in (MAXCH - 2, MAXCH - 1):
        ci = wid + ch * NW

        @pl.when(ci < N_CHUNKS)
        def _():
            pltpu.make_async_copy(out[ch % 2], p_slice(ci), so[ch % 2]).wait()


_sc_call = pl.kernel(
    _sc_body,
    out_type=jax.ShapeDtypeStruct((2, EB, 2, 128), jnp.float32),
    mesh=plsc.VectorSubcoreMesh(core_axis_name="c", subcore_axis_name="s"),
    compiler_params=pltpu.CompilerParams(
        needs_layout_passes=False, use_tc_tiling_on_sc=False
    ),
    scratch_types=[
        pltpu.VMEM((8, N_PAD), jnp.float32),
        pltpu.VMEM((BPC, 2, 128), jnp.int32),
        pltpu.VMEM((BPC, 2, 128), jnp.int32),
        pltpu.VMEM((2, BPC, 2, 128), jnp.float32),
        pltpu.VMEM((2, BPC, 2, 128), jnp.float32),
        pltpu.SemaphoreType.DMA,
        pltpu.SemaphoreType.DMA,
        pltpu.SemaphoreType.DMA,
        pltpu.SemaphoreType.DMA,
    ],
)


@jax.jit
def kernel(x, edge_index, W):
    table = _mm_call(x, W)
    q = jnp.transpose(edge_index.astype(jnp.int32).reshape(2, EB, 128), (1, 0, 2))
    p = _sc_call(table, q)
    return jnp.transpose(p, (1, 3, 0, 2)).reshape(N_EDGES, 2, 2)


# speedup vs baseline: 1.1325x; 1.1325x over previous
import jax
import jax.numpy as jnp
from jax import lax
from jax.experimental import pallas as pl
from jax.experimental.pallas import tpu as pltpu
from jax.experimental.pallas import tpu_sc as plsc

N_NODES = 10000
N_PAD = 10240
N_EDGES = 320000
D_FEAT = 128
EB = N_EDGES // 128

NUM_CORES = 2
NUM_SUBCORES = 16
NW = NUM_CORES * NUM_SUBCORES
BPC = 20
CHUNK = BPC * 128
N_CHUNKS = EB // BPC
_LN2 = 0.6931471805599453


def _mm_body(x_ref, w_ref, o_ref):
    xb = x_ref[:]
    w = w_ref[:]
    xp = jnp.concatenate(
        [xb, jnp.zeros((N_PAD - N_NODES, D_FEAT), jnp.float32)], axis=0
    )
    dn = (((1,), (1,)), ((), ()))
    ws = w * 2.0
    t1 = lax.dot_general(ws[:, :D_FEAT], xp, dn, preferred_element_type=jnp.float32)
    t2 = lax.dot_general(ws[:, D_FEAT:], xp, dn, preferred_element_type=jnp.float32)
    o_ref[:] = jnp.concatenate([t1, t2 - _LN2], axis=0)


_mm_call = pl.pallas_call(
    _mm_body,
    out_shape=jax.ShapeDtypeStruct((8, N_PAD), jnp.float32),
)


MAXCH = -(-N_CHUNKS // NW)


def _sc_body(
    table_hbm, q_hbm, p_hbm, table_v, idx0, idx1, out0, out1, si0, si1, so0, so1
):
    wid = lax.axis_index("s") * NUM_CORES + lax.axis_index("c")
    idx = (idx0, idx1)
    out = (out0, out1)
    si = (si0, si1)
    so = (so0, so1)

    pltpu.sync_copy(table_hbm, table_v)

    def q_slice(ci):
        return q_hbm.at[pl.ds(ci * BPC, BPC)]

    def p_slice(ci):
        return p_hbm.at[:, pl.ds(ci * BPC, BPC), :, :]

    def fire_idx(ci, b):
        @pl.when(ci < N_CHUNKS)
        def _():
            pltpu.async_copy(q_slice(ci), idx[b], si[b])

    def compute(b):
        @plsc.parallel_loop(0, CHUNK // 16, unroll=4)
        def _(g):
            eb = g // 8
            o = (g % 8) * 16
            r = idx[b][eb, 0, pl.ds(o, 16)]
            c = idx[b][eb, 1, pl.ds(o, 16)]
            for j in range(4):
                yj = plsc.load_gather(table_v, [jnp.full((16,), j, jnp.int32), r])
                zj = plsc.load_gather(table_v, [jnp.full((16,), j + 4, jnp.int32), c])
                s = yj + zj
                t = 1.0 - 1.0 / (jnp.exp(s) + 0.5)
                out[b][j // 2, eb, j % 2, pl.ds(o, 16)] = t

    fire_idx(wid, 0)
    fire_idx(wid + NW, 1)

    def pair_body(i, carry):
        for b in range(2):
            ch = 2 * i + b
            ci = wid + ch * NW
            ci_prev = ci - 2 * NW

            @pl.when(jnp.logical_and(ch >= 2, ci_prev < N_CHUNKS))
            def _():
                pltpu.make_async_copy(out[b], p_slice(ci_prev), so[b]).wait()

            @pl.when(ci < N_CHUNKS)
            def _():
                pltpu.make_async_copy(q_slice(ci), idx[b], si[b]).wait()
                compute(b)
                pltpu.async_copy(out[b], p_slice(ci), so[b])

            fire_idx(ci + 2 * NW, b)
        return carry

    lax.fori_loop(0, MAXCH // 2, pair_body, 0)

    for ch in (MAXCH - 2, MAXCH - 1):
        ci = wid + ch * NW

        @pl.when(ci < N_CHUNKS)
        def _():
            pltpu.make_async_copy(out[ch % 2], p_slice(ci), so[ch % 2]).wait()


_sc_call = pl.kernel(
    _sc_body,
    out_type=jax.ShapeDtypeStruct((2, EB, 2, 128), jnp.float32),
    mesh=plsc.VectorSubcoreMesh(core_axis_name="c", subcore_axis_name="s"),
    compiler_params=pltpu.CompilerParams(
        needs_layout_passes=False, use_tc_tiling_on_sc=False
    ),
    scratch_types=[
        pltpu.VMEM((8, N_PAD), jnp.float32),
        pltpu.VMEM((BPC, 2, 128), jnp.int32),
        pltpu.VMEM((BPC, 2, 128), jnp.int32),
        pltpu.VMEM((2, BPC, 2, 128), jnp.float32),
        pltpu.VMEM((2, BPC, 2, 128), jnp.float32),
        pltpu.SemaphoreType.DMA,
        pltpu.SemaphoreType.DMA,
        pltpu.SemaphoreType.DMA,
        pltpu.SemaphoreType.DMA,
    ],
)


@jax.jit
def kernel(x, edge_index, W):
    table = _mm_call(x, W)
    q = jnp.transpose(edge_index.astype(jnp.int32).reshape(2, EB, 128), (1, 0, 2))
    p = _sc_call(table, q)
    return jnp.transpose(p, (1, 3, 0, 2)).reshape(N_EDGES, 2, 2)


# speedup vs baseline: 1.1531x; 1.0182x over previous
import jax
import jax.numpy as jnp
from jax import lax
from jax.experimental import pallas as pl
from jax.experimental.pallas import tpu as pltpu
from jax.experimental.pallas import tpu_sc as plsc

N_NODES = 10000
N_PAD = 10240
N_EDGES = 320000
D_FEAT = 128
EB = N_EDGES // 128

NUM_CORES = 2
NUM_SUBCORES = 16
NW = NUM_CORES * NUM_SUBCORES
BPC = 20
CHUNK = BPC * 128
N_CHUNKS = EB // BPC
_LN2 = 0.6931471805599453


def _mm_body(x_ref, w_ref, o_ref):
    xb = x_ref[:]
    w = w_ref[:]
    xp = jnp.concatenate(
        [xb, jnp.zeros((N_PAD - N_NODES, D_FEAT), jnp.float32)], axis=0
    )
    dn = (((1,), (1,)), ((), ()))
    ws = w * 2.0
    t1 = lax.dot_general(ws[:, :D_FEAT], xp, dn, preferred_element_type=jnp.float32)
    t2 = lax.dot_general(ws[:, D_FEAT:], xp, dn, preferred_element_type=jnp.float32)
    o_ref[:] = jnp.concatenate([t1, t2 - _LN2], axis=0)


_mm_call = pl.pallas_call(
    _mm_body,
    out_shape=jax.ShapeDtypeStruct((8, N_PAD), jnp.float32),
)


MAXCH = -(-N_CHUNKS // NW)


def _sc_body(
    table_hbm, q_hbm, p_hbm, table_v, idx0, idx1, out0, out1, si0, si1, so0, so1
):
    wid = lax.axis_index("s") * NUM_CORES + lax.axis_index("c")
    idx = (idx0, idx1)
    out = (out0, out1)
    si = (si0, si1)
    so = (so0, so1)

    def q_slice(ci):
        return q_hbm.at[pl.ds(ci * BPC, BPC)]

    def p_slice(ci):
        return p_hbm.at[:, pl.ds(ci * BPC, BPC), :, :]

    def fire_idx(ci, b):
        @pl.when(ci < N_CHUNKS)
        def _():
            pltpu.async_copy(q_slice(ci), idx[b], si[b])

    def compute(b):
        @plsc.parallel_loop(0, CHUNK // 16, unroll=4)
        def _(g):
            eb = g // 8
            o = (g % 8) * 16
            r = idx[b][eb, 0, pl.ds(o, 16)]
            c = idx[b][eb, 1, pl.ds(o, 16)]
            for j in range(4):
                yj = plsc.load_gather(table_v, [jnp.full((16,), j, jnp.int32), r])
                zj = plsc.load_gather(table_v, [jnp.full((16,), j + 4, jnp.int32), c])
                s = yj + zj
                t = 1.0 - 1.0 / (jnp.exp(s) + 0.5)
                out[b][j // 2, eb, j % 2, pl.ds(o, 16)] = t

    fire_idx(wid, 0)
    fire_idx(wid + NW, 1)
    pltpu.sync_copy(table_hbm, table_v)

    def pair_body(i, carry):
        for b in range(2):
            ch = 2 * i + b
            ci = wid + ch * NW
            ci_prev = ci - 2 * NW

            @pl.when(jnp.logical_and(ch >= 2, ci_prev < N_CHUNKS))
            def _():
                pltpu.make_async_copy(out[b], p_slice(ci_prev), so[b]).wait()

            @pl.when(ci < N_CHUNKS)
            def _():
                pltpu.make_async_copy(q_slice(ci), idx[b], si[b]).wait()
                compute(b)
                pltpu.async_copy(out[b], p_slice(ci), so[b])

            fire_idx(ci + 2 * NW, b)
        return carry

    lax.fori_loop(0, MAXCH // 2, pair_body, 0)

    for ch in (MAXCH - 2, MAXCH - 1):
        ci = wid + ch * NW

        @pl.when(ci < N_CHUNKS)
        def _():
            pltpu.make_async_copy(out[ch % 2], p_slice(ci), so[ch % 2]).wait()


_sc_call = pl.kernel(
    _sc_body,
    out_type=jax.ShapeDtypeStruct((2, EB, 2, 128), jnp.float32),
    mesh=plsc.VectorSubcoreMesh(core_axis_name="c", subcore_axis_name="s"),
    compiler_params=pltpu.CompilerParams(
        needs_layout_passes=False, use_tc_tiling_on_sc=False
    ),
    scratch_types=[
        pltpu.VMEM((8, N_PAD), jnp.float32),
        pltpu.VMEM((BPC, 2, 128), jnp.int32),
        pltpu.VMEM((BPC, 2, 128), jnp.int32),
        pltpu.VMEM((2, BPC, 2, 128), jnp.float32),
        pltpu.VMEM((2, BPC, 2, 128), jnp.float32),
        pltpu.SemaphoreType.DMA,
        pltpu.SemaphoreType.DMA,
        pltpu.SemaphoreType.DMA,
        pltpu.SemaphoreType.DMA,
    ],
)


@jax.jit
def kernel(x, edge_index, W):
    table = _mm_call(x, W)
    q = jnp.transpose(edge_index.astype(jnp.int32).reshape(2, EB, 128), (1, 0, 2))
    p = _sc_call(table, q)
    return jnp.transpose(p, (1, 3, 0, 2)).reshape(N_EDGES, 2, 2)
